# baseline (speedup 1.0000x reference)
import jax
import jax.numpy as jnp
from jax import lax
from jax.experimental import pallas as pl
from jax.experimental.pallas import tpu as pltpu
from jax.experimental.pallas import tpu_sc as plsc

D = 128
DEG = 32
L = 16
NC = 2
NS = 16
NW = NC * NS
CHUNK = 16
CD = D // L


def _proj_body(f_ref, w1_ref, re_ref, w2_ref, ba_ref, adj_ref, rat_ref,
               p_ref, rp_ref, ari_ref):
    p_ref[...] = jnp.dot(f_ref[...], w1_ref[...],
                         preferred_element_type=jnp.float32)
    rp_ref[...] = jnp.dot(re_ref[...], w2_ref[...],
                          preferred_element_type=jnp.float32) + ba_ref[...]
    blk = adj_ref.shape[0]
    ari_ref[...] = jnp.concatenate(
        [adj_ref[...], rat_ref[...],
         jnp.zeros((blk, D - 2 * DEG), jnp.int32)], axis=1)


def _project_table(features, Wa1, re_pad, Wa2, b_agg, adj, ratings):
    N = features.shape[0]
    BLK = 1000
    grid = N // BLK
    return pl.pallas_call(
        _proj_body,
        grid=(grid,),
        in_specs=[
            pl.BlockSpec((BLK, D), lambda i: (i, 0)),
            pl.BlockSpec((D, D), lambda i: (0, 0)),
            pl.BlockSpec((8, D), lambda i: (0, 0)),
            pl.BlockSpec((D, D), lambda i: (0, 0)),
            pl.BlockSpec((1, D), lambda i: (0, 0)),
            pl.BlockSpec((BLK, DEG), lambda i: (i, 0)),
            pl.BlockSpec((BLK, DEG), lambda i: (i, 0)),
        ],
        out_specs=[
            pl.BlockSpec((BLK, D), lambda i: (i, 0)),
            pl.BlockSpec((8, D), lambda i: (0, 0)),
            pl.BlockSpec((BLK, D), lambda i: (i, 0)),
        ],
        out_shape=[
            jax.ShapeDtypeStruct((N, D), jnp.float32),
            jax.ShapeDtypeStruct((8, D), jnp.float32),
            jax.ShapeDtypeStruct((N, D), jnp.int32),
        ],
    )(features, Wa1, re_pad, Wa2, b_agg, adj, ratings)


GPC = 128 // DEG
NG = CHUNK // GPC


def _sc_body(nodes_h, ari_h, feat_h, p_h, rp_h,
             selfo_h, neigho_h,
             nodes_v, ari_v, idxp_v, idxq_v, prow_v, qrow_v, selfb_v, neigh_v,
             rpst_v, rp_sh,
             sem_ari, sem_self, sem_p0, sem_p1, sem_q0, sem_q1):
    wid = lax.axis_index("s") * NC + lax.axis_index("c")
    bpw = nodes_h.shape[0] // NW
    base = wid * bpw
    nchunk = bpw // CHUNK
    psems = (sem_p0, sem_p1)
    qsems = (sem_q0, sem_q1)

    def p_copy(g, par):
        return pltpu.make_async_copy(
            p_h.at[idxp_v.at[g]], prow_v.at[par], psems[par])

    def q_copy(g, par):
        return pltpu.make_async_copy(
            rp_h.at[idxq_v.at[g]], qrow_v.at[par], qsems[par])

    def chunk_body(j, _):
        cb = base + j * CHUNK
        pltpu.sync_copy(nodes_h.at[pl.ds(cb, CHUNK)], nodes_v)
        cp_ari = pltpu.make_async_copy(ari_h.at[nodes_v], ari_v, sem_ari)
        cp_ari.start()
        cp_self = pltpu.make_async_copy(feat_h.at[nodes_v], selfb_v, sem_self)
        cp_self.start()
        cp_ari.wait()

        iota = lax.iota(jnp.int32, L)
        for i in range(CHUNK):
            g, s = i // GPC, (i % GPC) * DEG
            for hv in range(DEG // L):
                idxp_v[g, pl.ds(s + hv * L, L)] = \
                    ari_v[i, pl.ds(hv * L, L)]
                idxq_v[g, pl.ds(s + hv * L, L)] = \
                    ari_v[i, pl.ds(DEG + hv * L, L)] * DEG + (hv * L + iota)

        p_copy(0, 0).start()
        q_copy(0, 0).start()

        def compute_group(g, par):
            for n in range(GPC):
                i = g * GPC + n
                accs = tuple(jnp.zeros((L,), jnp.float32) for _ in range(CD))

                def k_body(k, accs, n=n):
                    row = n * DEG + k
                    out = []
                    for c in range(CD):
                        pv = prow_v[par, row, pl.ds(c * L, L)]
                        qv = qrow_v[par, row, pl.ds(c * L, L)]
                        out.append(accs[c] + jnp.maximum(pv + qv, 0.0))
                    return tuple(out)

                accs = lax.fori_loop(0, DEG, k_body, accs)
                for c in range(CD):
                    neigh_v[i, pl.ds(c * L, L)] = accs[c]

        def pair_body(p, _):
            for par in range(2):
                g = p * 2 + par

                @pl.when(g + 1 < NG)
                def _():
                    p_copy(g + 1, (par + 1) % 2).start()
                    q_copy(g + 1, (par + 1) % 2).start()

                p_copy(g, par).wait()
                q_copy(g, par).wait()
                compute_group(g, par)
            return 0

        lax.fori_loop(0, NG // 2, pair_body, 0)

        cp_self.wait()
        pltpu.sync_copy(selfb_v, selfo_h.at[pl.ds(cb, CHUNK)])
        pltpu.sync_copy(neigh_v, neigho_h.at[pl.ds(cb, CHUNK)])
        return 0

    lax.fori_loop(0, nchunk, chunk_body, 0)


def _sc_gather_reduce(nodes_pad, ari, features, P, rp):
    BP = nodes_pad.shape[0]
    mesh = plsc.VectorSubcoreMesh(
        core_axis_name="c", subcore_axis_name="s",
        num_cores=NC, num_subcores=NS)
    f = pl.kernel(
        _sc_body,
        out_type=[
            jax.ShapeDtypeStruct((BP, D), jnp.float32),
            jax.ShapeDtypeStruct((BP, D), jnp.float32),
        ],
        mesh=mesh,
        scratch_types=[
            pltpu.VMEM((CHUNK,), jnp.int32),
            pltpu.VMEM((CHUNK, D), jnp.int32),
            pltpu.VMEM((NG, 128), jnp.int32),
            pltpu.VMEM((NG, 128), jnp.int32),
            pltpu.VMEM((2, GPC * DEG, D), jnp.float32),
            pltpu.VMEM((2, GPC * DEG, D), jnp.float32),
            pltpu.VMEM((CHUNK, D), jnp.float32),
            pltpu.VMEM((CHUNK, D), jnp.float32),
            pltpu.VMEM((5 * DEG, D), jnp.float32),
            pltpu.VMEM_SHARED((5 * DEG, D), jnp.float32),
            pltpu.SemaphoreType.DMA,
            pltpu.SemaphoreType.DMA,
            pltpu.SemaphoreType.DMA,
            pltpu.SemaphoreType.DMA,
            pltpu.SemaphoreType.DMA,
            pltpu.SemaphoreType.DMA,
        ],
    )
    return f(nodes_pad, ari, features, P, rp)


def _final_body(s_ref, n_ref, w1_ref, w2_ref, b_ref, o_ref):
    acc = jnp.dot(s_ref[...], w1_ref[...], preferred_element_type=jnp.float32)
    acc = acc + jnp.dot(n_ref[...], w2_ref[...],
                        preferred_element_type=jnp.float32)
    o_ref[...] = jnp.maximum(acc + b_ref[...], 0.0)


def _final_linear(selfF, neighS, W1, W2s, b2d):
    BP = selfF.shape[0]
    BLK = 1280
    grid = BP // BLK
    return pl.pallas_call(
        _final_body,
        grid=(grid,),
        in_specs=[
            pl.BlockSpec((BLK, D), lambda i: (i, 0)),
            pl.BlockSpec((BLK, D), lambda i: (i, 0)),
            pl.BlockSpec((D, D), lambda i: (0, 0)),
            pl.BlockSpec((D, D), lambda i: (0, 0)),
            pl.BlockSpec((1, D), lambda i: (0, 0)),
        ],
        out_specs=pl.BlockSpec((BLK, D), lambda i: (i, 0)),
        out_shape=jax.ShapeDtypeStruct((BP, D), jnp.float32),
    )(selfF, neighS, W1, W2s, b2d)


@jax.jit
def kernel(nodes, adj, ratings, features, rating_embed, W_agg, b_agg, W, b):
    B = nodes.shape[0]
    BP = ((B + 8 * NW - 1) // (8 * NW)) * (8 * NW)
    nodes_pad = jnp.pad(nodes.astype(jnp.int32), (0, BP - B))
    re_pad = jnp.pad(rating_embed, ((0, 8 - rating_embed.shape[0]), (0, 0)))

    P, rp, ari = _project_table(features, W_agg[:D], re_pad, W_agg[D:],
                                b_agg.reshape(1, D),
                                adj.astype(jnp.int32),
                                ratings.astype(jnp.int32))
    rp_rep = jnp.broadcast_to(rp[:5, None, :], (5, DEG, D)).reshape(5 * DEG, D)
    selfF, neighS = _sc_gather_reduce(nodes_pad, ari, features, P, rp_rep)
    out = _final_linear(selfF, neighS, W[:D], W[D:] * (1.0 / DEG),
                        b.reshape(1, D))
    return out[:B]


# speedup vs baseline: 1.3305x; 1.3305x over previous
import jax
import jax.numpy as jnp
from jax import lax
from jax.experimental import pallas as pl
from jax.experimental.pallas import tpu as pltpu
from jax.experimental.pallas import tpu_sc as plsc

D = 128
DEG = 32
L = 16
NC = 2
NS = 16
NW = NC * NS
CHUNK = 16
CD = D // L


def _proj_body(f_ref, w1_ref, re_ref, w2_ref, ba_ref, adj_ref, rat_ref,
               p_ref, rp_ref, ari_ref):
    p_ref[...] = jnp.dot(f_ref[...], w1_ref[...],
                         preferred_element_type=jnp.float32)
    rp_ref[...] = jnp.dot(re_ref[...], w2_ref[...],
                          preferred_element_type=jnp.float32) + ba_ref[...]
    blk = adj_ref.shape[0]
    ari_ref[...] = jnp.concatenate(
        [adj_ref[...], rat_ref[...],
         jnp.zeros((blk, D - 2 * DEG), jnp.int32)], axis=1)


def _project_table(features, Wa1, re_pad, Wa2, b_agg, adj, ratings):
    N = features.shape[0]
    BLK = 1000
    grid = N // BLK
    return pl.pallas_call(
        _proj_body,
        grid=(grid,),
        in_specs=[
            pl.BlockSpec((BLK, D), lambda i: (i, 0)),
            pl.BlockSpec((D, D), lambda i: (0, 0)),
            pl.BlockSpec((8, D), lambda i: (0, 0)),
            pl.BlockSpec((D, D), lambda i: (0, 0)),
            pl.BlockSpec((1, D), lambda i: (0, 0)),
            pl.BlockSpec((BLK, DEG), lambda i: (i, 0)),
            pl.BlockSpec((BLK, DEG), lambda i: (i, 0)),
        ],
        out_specs=[
            pl.BlockSpec((BLK, D), lambda i: (i, 0)),
            pl.BlockSpec((8, D), lambda i: (0, 0)),
            pl.BlockSpec((BLK, D), lambda i: (i, 0)),
        ],
        out_shape=[
            jax.ShapeDtypeStruct((N, D), jnp.float32),
            jax.ShapeDtypeStruct((8, D), jnp.float32),
            jax.ShapeDtypeStruct((N, D), jnp.int32),
        ],
    )(features, Wa1, re_pad, Wa2, b_agg, adj, ratings)


GPC = 128 // DEG
NG = CHUNK // GPC


def _sc_body(nodes_h, ari_h, feat_h, p_h, rp_h,
             selfo_h, neigho_h,
             nodes_v, ari_v, idxp_v, idxq_v, prow_v, qrow_v, selfb_v, neigh_v,
             rpst_v, rp_sh,
             sem_ari, sem_self, sem_p0, sem_p1, sem_q0, sem_q1):
    wid = lax.axis_index("s") * NC + lax.axis_index("c")
    bpw = nodes_h.shape[0] // NW
    base = wid * bpw
    nchunk = bpw // CHUNK
    psems = (sem_p0, sem_p1)
    qsems = (sem_q0, sem_q1)
    cid = lax.axis_index("c")

    pltpu.sync_copy(rp_h, rpst_v)
    pltpu.sync_copy(rpst_v, rp_sh.at[cid])

    def p_copy(g, par):
        return pltpu.make_async_copy(
            p_h.at[idxp_v.at[g]], prow_v.at[par], psems[par])

    def q_copy(g, par):
        return pltpu.make_async_copy(
            rp_sh.at[cid].at[idxq_v.at[g]], qrow_v.at[par], qsems[par])

    def chunk_body(j, _):
        cb = base + j * CHUNK
        pltpu.sync_copy(nodes_h.at[pl.ds(cb, CHUNK)], nodes_v)
        cp_ari = pltpu.make_async_copy(ari_h.at[nodes_v], ari_v, sem_ari)
        cp_ari.start()
        cp_self = pltpu.make_async_copy(feat_h.at[nodes_v], selfb_v, sem_self)
        cp_self.start()
        cp_ari.wait()

        iota = lax.iota(jnp.int32, L)
        for i in range(CHUNK):
            g, s = i // GPC, (i % GPC) * DEG
            for hv in range(DEG // L):
                idxp_v[g, pl.ds(s + hv * L, L)] = \
                    ari_v[i, pl.ds(hv * L, L)]
                idxq_v[g, pl.ds(s + hv * L, L)] = \
                    ari_v[i, pl.ds(DEG + hv * L, L)] * DEG + (hv * L + iota)

        p_copy(0, 0).start()
        q_copy(0, 0).start()

        def compute_group(g, par):
            for n in range(GPC):
                i = g * GPC + n
                accs = tuple(jnp.zeros((L,), jnp.float32) for _ in range(CD))

                def k_body(k, accs, n=n):
                    row = n * DEG + k
                    out = []
                    for c in range(CD):
                        pv = prow_v[par, row, pl.ds(c * L, L)]
                        qv = qrow_v[par, row, pl.ds(c * L, L)]
                        out.append(accs[c] + jnp.maximum(pv + qv, 0.0))
                    return tuple(out)

                accs = lax.fori_loop(0, DEG, k_body, accs)
                for c in range(CD):
                    neigh_v[i, pl.ds(c * L, L)] = accs[c]

        def pair_body(p, _):
            for par in range(2):
                g = p * 2 + par

                @pl.when(g + 1 < NG)
                def _():
                    p_copy(g + 1, (par + 1) % 2).start()
                    q_copy(g + 1, (par + 1) % 2).start()

                p_copy(g, par).wait()
                q_copy(g, par).wait()
                compute_group(g, par)
            return 0

        lax.fori_loop(0, NG // 2, pair_body, 0)

        cp_self.wait()
        pltpu.sync_copy(selfb_v, selfo_h.at[pl.ds(cb, CHUNK)])
        pltpu.sync_copy(neigh_v, neigho_h.at[pl.ds(cb, CHUNK)])
        return 0

    lax.fori_loop(0, nchunk, chunk_body, 0)


def _sc_gather_reduce(nodes_pad, ari, features, P, rp):
    BP = nodes_pad.shape[0]
    mesh = plsc.VectorSubcoreMesh(
        core_axis_name="c", subcore_axis_name="s",
        num_cores=NC, num_subcores=NS)
    f = pl.kernel(
        _sc_body,
        out_type=[
            jax.ShapeDtypeStruct((BP, D), jnp.float32),
            jax.ShapeDtypeStruct((BP, D), jnp.float32),
        ],
        mesh=mesh,
        scratch_types=[
            pltpu.VMEM((CHUNK,), jnp.int32),
            pltpu.VMEM((CHUNK, D), jnp.int32),
            pltpu.VMEM((NG, 128), jnp.int32),
            pltpu.VMEM((NG, 128), jnp.int32),
            pltpu.VMEM((2, GPC * DEG, D), jnp.float32),
            pltpu.VMEM((2, GPC * DEG, D), jnp.float32),
            pltpu.VMEM((CHUNK, D), jnp.float32),
            pltpu.VMEM((CHUNK, D), jnp.float32),
            pltpu.VMEM((5 * DEG, D), jnp.float32),
            pltpu.VMEM_SHARED((NC, 5 * DEG, D), jnp.float32),
            pltpu.SemaphoreType.DMA,
            pltpu.SemaphoreType.DMA,
            pltpu.SemaphoreType.DMA,
            pltpu.SemaphoreType.DMA,
            pltpu.SemaphoreType.DMA,
            pltpu.SemaphoreType.DMA,
        ],
    )
    return f(nodes_pad, ari, features, P, rp)


def _final_body(s_ref, n_ref, w1_ref, w2_ref, b_ref, o_ref):
    acc = jnp.dot(s_ref[...], w1_ref[...], preferred_element_type=jnp.float32)
    acc = acc + jnp.dot(n_ref[...], w2_ref[...],
                        preferred_element_type=jnp.float32)
    o_ref[...] = jnp.maximum(acc + b_ref[...], 0.0)


def _final_linear(selfF, neighS, W1, W2s, b2d):
    BP = selfF.shape[0]
    BLK = 1280
    grid = BP // BLK
    return pl.pallas_call(
        _final_body,
        grid=(grid,),
        in_specs=[
            pl.BlockSpec((BLK, D), lambda i: (i, 0)),
            pl.BlockSpec((BLK, D), lambda i: (i, 0)),
            pl.BlockSpec((D, D), lambda i: (0, 0)),
            pl.BlockSpec((D, D), lambda i: (0, 0)),
            pl.BlockSpec((1, D), lambda i: (0, 0)),
        ],
        out_specs=pl.BlockSpec((BLK, D), lambda i: (i, 0)),
        out_shape=jax.ShapeDtypeStruct((BP, D), jnp.float32),
    )(selfF, neighS, W1, W2s, b2d)


@jax.jit
def kernel(nodes, adj, ratings, features, rating_embed, W_agg, b_agg, W, b):
    B = nodes.shape[0]
    BP = ((B + 8 * NW - 1) // (8 * NW)) * (8 * NW)
    nodes_pad = jnp.pad(nodes.astype(jnp.int32), (0, BP - B))
    re_pad = jnp.pad(rating_embed, ((0, 8 - rating_embed.shape[0]), (0, 0)))

    P, rp, ari = _project_table(features, W_agg[:D], re_pad, W_agg[D:],
                                b_agg.reshape(1, D),
                                adj.astype(jnp.int32),
                                ratings.astype(jnp.int32))
    rp_rep = jnp.broadcast_to(rp[:5, None, :], (5, DEG, D)).reshape(5 * DEG, D)
    selfF, neighS = _sc_gather_reduce(nodes_pad, ari, features, P, rp_rep)
    out = _final_linear(selfF, neighS, W[:D], W[D:] * (1.0 / DEG),
                        b.reshape(1, D))
    return out[:B]


# speedup vs baseline: 1.4867x; 1.1174x over previous
import jax
import jax.numpy as jnp
from jax import lax
from jax.experimental import pallas as pl
from jax.experimental.pallas import tpu as pltpu
from jax.experimental.pallas import tpu_sc as plsc

D = 128
DEG = 32
L = 16
NC = 2
NS = 16
NW = NC * NS
CHUNK = 64
CD = D // L


def _proj_body(f_ref, w1_ref, re_ref, w2_ref, ba_ref, adj_ref, rat_ref,
               p_ref, rp_ref, ari_ref):
    p_ref[...] = jnp.dot(f_ref[...], w1_ref[...],
                         preferred_element_type=jnp.float32)
    rp_ref[...] = jnp.dot(re_ref[...], w2_ref[...],
                          preferred_element_type=jnp.float32) + ba_ref[...]
    blk = adj_ref.shape[0]
    ari_ref[...] = jnp.concatenate(
        [adj_ref[...], rat_ref[...],
         jnp.zeros((blk, D - 2 * DEG), jnp.int32)], axis=1)


def _project_table(features, Wa1, re_pad, Wa2, b_agg, adj, ratings):
    N = features.shape[0]
    BLK = 1000
    grid = N // BLK
    return pl.pallas_call(
        _proj_body,
        grid=(grid,),
        in_specs=[
            pl.BlockSpec((BLK, D), lambda i: (i, 0)),
            pl.BlockSpec((D, D), lambda i: (0, 0)),
            pl.BlockSpec((8, D), lambda i: (0, 0)),
            pl.BlockSpec((D, D), lambda i: (0, 0)),
            pl.BlockSpec((1, D), lambda i: (0, 0)),
            pl.BlockSpec((BLK, DEG), lambda i: (i, 0)),
            pl.BlockSpec((BLK, DEG), lambda i: (i, 0)),
        ],
        out_specs=[
            pl.BlockSpec((BLK, D), lambda i: (i, 0)),
            pl.BlockSpec((8, D), lambda i: (0, 0)),
            pl.BlockSpec((BLK, D), lambda i: (i, 0)),
        ],
        out_shape=[
            jax.ShapeDtypeStruct((N, D), jnp.float32),
            jax.ShapeDtypeStruct((8, D), jnp.float32),
            jax.ShapeDtypeStruct((N, D), jnp.int32),
        ],
    )(features, Wa1, re_pad, Wa2, b_agg, adj, ratings)


GPC = 128 // DEG
NG = CHUNK // GPC


def _sc_body(nodes_h, ari_h, feat_h, p_h, rp_h,
             selfo_h, neigho_h,
             nodes_v, ari_v, idxp_v, idxq_v, prow_v, qrow_v, selfb_v, neigh_v,
             rp_sh,
             sem_ari, sem_self, sem_p0, sem_p1, sem_q0, sem_q1):
    wid = lax.axis_index("s") * NC + lax.axis_index("c")
    bpw = nodes_h.shape[0] // NW
    base = wid * bpw
    nchunk = bpw // CHUNK
    psems = (sem_p0, sem_p1)
    qsems = (sem_q0, sem_q1)
    cid = lax.axis_index("c")

    pltpu.sync_copy(rp_h.at[pl.ds(0, 128)], prow_v.at[0])
    pltpu.sync_copy(rp_h.at[pl.ds(128, 32)], prow_v.at[1, pl.ds(0, 32)])
    pltpu.sync_copy(prow_v.at[0], rp_sh.at[cid, pl.ds(0, 128)])
    pltpu.sync_copy(prow_v.at[1, pl.ds(0, 32)], rp_sh.at[cid, pl.ds(128, 32)])

    def p_copy(g, par):
        return pltpu.make_async_copy(
            p_h.at[idxp_v.at[g]], prow_v.at[par], psems[par])

    def q_copy(g, par):
        return pltpu.make_async_copy(
            rp_sh.at[cid].at[idxq_v.at[g]], qrow_v.at[par], qsems[par])

    def chunk_body(j, _):
        cb = base + j * CHUNK
        pltpu.sync_copy(nodes_h.at[pl.ds(cb, CHUNK)], nodes_v)
        cp_ari = pltpu.make_async_copy(ari_h.at[nodes_v], ari_v, sem_ari)
        cp_ari.start()
        cp_self = pltpu.make_async_copy(feat_h.at[nodes_v], selfb_v, sem_self)
        cp_self.start()
        cp_ari.wait()

        iota = lax.iota(jnp.int32, L)
        for i in range(CHUNK):
            g, s = i // GPC, (i % GPC) * DEG
            for hv in range(DEG // L):
                idxp_v[g, pl.ds(s + hv * L, L)] = \
                    ari_v[i, pl.ds(hv * L, L)]
                idxq_v[g, pl.ds(s + hv * L, L)] = \
                    ari_v[i, pl.ds(DEG + hv * L, L)] * DEG + (hv * L + iota)

        p_copy(0, 0).start()
        q_copy(0, 0).start()

        def compute_group(g, par):
            for n in range(GPC):
                i = g * GPC + n
                accs = tuple(jnp.zeros((L,), jnp.float32) for _ in range(CD))

                def k_body(k, accs, n=n):
                    row = n * DEG + k
                    out = []
                    for c in range(CD):
                        pv = prow_v[par, row, pl.ds(c * L, L)]
                        qv = qrow_v[par, row, pl.ds(c * L, L)]
                        out.append(accs[c] + jnp.maximum(pv + qv, 0.0))
                    return tuple(out)

                accs = lax.fori_loop(0, DEG, k_body, accs)
                for c in range(CD):
                    neigh_v[i, pl.ds(c * L, L)] = accs[c]

        def pair_body(p, _):
            for par in range(2):
                g = p * 2 + par

                @pl.when(g + 1 < NG)
                def _():
                    p_copy(g + 1, (par + 1) % 2).start()
                    q_copy(g + 1, (par + 1) % 2).start()

                p_copy(g, par).wait()
                q_copy(g, par).wait()
                compute_group(g, par)
            return 0

        lax.fori_loop(0, NG // 2, pair_body, 0)

        cp_self.wait()
        pltpu.sync_copy(selfb_v, selfo_h.at[pl.ds(cb, CHUNK)])
        pltpu.sync_copy(neigh_v, neigho_h.at[pl.ds(cb, CHUNK)])
        return 0

    lax.fori_loop(0, nchunk, chunk_body, 0)


def _sc_gather_reduce(nodes_pad, ari, features, P, rp):
    BP = nodes_pad.shape[0]
    mesh = plsc.VectorSubcoreMesh(
        core_axis_name="c", subcore_axis_name="s",
        num_cores=NC, num_subcores=NS)
    f = pl.kernel(
        _sc_body,
        out_type=[
            jax.ShapeDtypeStruct((BP, D), jnp.float32),
            jax.ShapeDtypeStruct((BP, D), jnp.float32),
        ],
        mesh=mesh,
        scratch_types=[
            pltpu.VMEM((CHUNK,), jnp.int32),
            pltpu.VMEM((CHUNK, D), jnp.int32),
            pltpu.VMEM((NG, 128), jnp.int32),
            pltpu.VMEM((NG, 128), jnp.int32),
            pltpu.VMEM((2, GPC * DEG, D), jnp.float32),
            pltpu.VMEM((2, GPC * DEG, D), jnp.float32),
            pltpu.VMEM((CHUNK, D), jnp.float32),
            pltpu.VMEM((CHUNK, D), jnp.float32),
            pltpu.VMEM_SHARED((NC, 5 * DEG, D), jnp.float32),
            pltpu.SemaphoreType.DMA,
            pltpu.SemaphoreType.DMA,
            pltpu.SemaphoreType.DMA,
            pltpu.SemaphoreType.DMA,
            pltpu.SemaphoreType.DMA,
            pltpu.SemaphoreType.DMA,
        ],
    )
    return f(nodes_pad, ari, features, P, rp)


def _final_body(s_ref, n_ref, w1_ref, w2_ref, b_ref, o_ref):
    acc = jnp.dot(s_ref[...], w1_ref[...], preferred_element_type=jnp.float32)
    acc = acc + jnp.dot(n_ref[...], w2_ref[...],
                        preferred_element_type=jnp.float32)
    o_ref[...] = jnp.maximum(acc + b_ref[...], 0.0)


def _final_linear(selfF, neighS, W1, W2s, b2d):
    BP = selfF.shape[0]
    BLK = 1280
    grid = BP // BLK
    return pl.pallas_call(
        _final_body,
        grid=(grid,),
        in_specs=[
            pl.BlockSpec((BLK, D), lambda i: (i, 0)),
            pl.BlockSpec((BLK, D), lambda i: (i, 0)),
            pl.BlockSpec((D, D), lambda i: (0, 0)),
            pl.BlockSpec((D, D), lambda i: (0, 0)),
            pl.BlockSpec((1, D), lambda i: (0, 0)),
        ],
        out_specs=pl.BlockSpec((BLK, D), lambda i: (i, 0)),
        out_shape=jax.ShapeDtypeStruct((BP, D), jnp.float32),
    )(selfF, neighS, W1, W2s, b2d)


@jax.jit
def kernel(nodes, adj, ratings, features, rating_embed, W_agg, b_agg, W, b):
    B = nodes.shape[0]
    BP = ((B + 8 * NW - 1) // (8 * NW)) * (8 * NW)
    nodes_pad = jnp.pad(nodes.astype(jnp.int32), (0, BP - B))
    re_pad = jnp.pad(rating_embed, ((0, 8 - rating_embed.shape[0]), (0, 0)))

    P, rp, ari = _project_table(features, W_agg[:D], re_pad, W_agg[D:],
                                b_agg.reshape(1, D),
                                adj.astype(jnp.int32),
                                ratings.astype(jnp.int32))
    rp_rep = jnp.broadcast_to(rp[:5, None, :], (5, DEG, D)).reshape(5 * DEG, D)
    selfF, neighS = _sc_gather_reduce(nodes_pad, ari, features, P, rp_rep)
    out = _final_linear(selfF, neighS, W[:D], W[D:] * (1.0 / DEG),
                        b.reshape(1, D))
    return out[:B]
